# baseline (speedup 1.0000x reference)
import jax
import jax.numpy as jnp
from jax.experimental import pallas as pl

_NUM_CB = 4


def _vq_block(z_ref, cb_ref, cbm2_ref, wsq_ref, zq_ref, idx_ref, loss_ref):
    zb = z_ref[0]
    C, NT = zb.shape
    ncb, K, dpc = cb_ref.shape
    acc = jnp.zeros((), jnp.float32)
    for i in range(ncb):
        zi = zb[dpc * i:dpc * (i + 1), :]
        wsq = wsq_ref[i]
        zsq = jnp.sum(zi * zi, axis=0, keepdims=True)
        prod = jax.lax.dot_general(
            cbm2_ref[i], zi, (((1,), (0,)), ((), ())),
            preferred_element_type=jnp.float32,
            precision=jax.lax.Precision.DEFAULT)
        d = (zsq + wsq) + prod
        m = jnp.min(d, axis=0, keepdims=True)
        row = jax.lax.broadcasted_iota(jnp.int32, d.shape, 0)
        idx = jnp.min(jnp.where(d == m, row, K), axis=0, keepdims=True)
        onehot = (row == idx).astype(jnp.bfloat16)
        zq = jax.lax.dot_general(
            cb_ref[i], onehot, (((0,), (0,)), ((), ())),
            preferred_element_type=jnp.float32,
            precision=jax.lax.Precision.DEFAULT)
        zq_ref[0, dpc * i:dpc * (i + 1), :] = zq
        idx_ref[i:i + 1, :] = idx
        acc = acc + jnp.sum(m)

    first = jnp.logical_and(pl.program_id(0) == 0, pl.program_id(1) == 0)
    acc2 = acc.reshape(1, 1)

    @pl.when(first)
    def _():
        loss_ref[:, :] = acc2

    @pl.when(jnp.logical_not(first))
    def _():
        loss_ref[:, :] = loss_ref[:, :] + acc2


def kernel(z, codebooks):
    b, c, h, w = z.shape
    n = h * w
    ncb, K, dpc = codebooks.shape
    z3 = z.reshape(b, c, n)
    cbm2 = codebooks * (-2.0)
    wsq = jnp.sum(codebooks * codebooks, axis=2)[:, :, None]
    NT = 4096 if n % 4096 == 0 else n
    tpb = n // NT
    grid = (b, tpb)
    zq3, idx2, loss_acc = pl.pallas_call(
        _vq_block,
        grid=grid,
        in_specs=[
            pl.BlockSpec((1, c, NT), lambda bi, ti: (bi, 0, ti)),
            pl.BlockSpec((ncb, K, dpc), lambda bi, ti: (0, 0, 0)),
            pl.BlockSpec((ncb, K, dpc), lambda bi, ti: (0, 0, 0)),
            pl.BlockSpec((ncb, K, 1), lambda bi, ti: (0, 0, 0)),
        ],
        out_specs=[
            pl.BlockSpec((1, c, NT), lambda bi, ti: (bi, 0, ti)),
            pl.BlockSpec((ncb, NT), lambda bi, ti: (0, bi * tpb + ti)),
            pl.BlockSpec((1, 1), lambda bi, ti: (0, 0)),
        ],
        out_shape=[
            jax.ShapeDtypeStruct((b, c, n), jnp.float32),
            jax.ShapeDtypeStruct((ncb, b * n), jnp.int32),
            jax.ShapeDtypeStruct((1, 1), jnp.float32),
        ],
    )(z3, codebooks, cbm2, wsq)
    z_q = zq3.reshape(b, c, h, w)
    total_loss = loss_acc[0, 0] * (1.25 / (ncb * b * n * dpc))
    indices = tuple(idx2[i].reshape(b, h, w) for i in range(ncb))
    return (z_q, total_loss, *indices)


# speedup vs baseline: 1.0065x; 1.0065x over previous
import jax
import jax.numpy as jnp
from jax.experimental import pallas as pl

_NUM_CB = 4


def _vq_block(z_ref, cb_ref, cbm2_ref, wsq_ref, zq_ref, idx_ref, loss_ref):
    zb = z_ref[0]
    C, NT = zb.shape
    ncb, K, dpc = cb_ref.shape
    acc = jnp.zeros((), jnp.float32)
    for i in range(ncb):
        zi = zb[dpc * i:dpc * (i + 1), :]
        wsq = wsq_ref[i]
        zsq = jnp.sum(zi * zi, axis=0, keepdims=True)
        prod = jax.lax.dot_general(
            cbm2_ref[i], zi, (((1,), (0,)), ((), ())),
            preferred_element_type=jnp.float32,
            precision=jax.lax.Precision.DEFAULT)
        d = (zsq + wsq) + prod
        m = jnp.min(d, axis=0, keepdims=True)
        row = jax.lax.broadcasted_iota(jnp.int32, d.shape, 0)
        idx = jnp.min(jnp.where(d == m, row, K), axis=0, keepdims=True)
        onehot = (row == idx).astype(jnp.bfloat16)
        zq = jax.lax.dot_general(
            cb_ref[i], onehot, (((0,), (0,)), ((), ())),
            preferred_element_type=jnp.float32,
            precision=jax.lax.Precision.DEFAULT)
        zq_ref[0, dpc * i:dpc * (i + 1), :] = zq
        idx_ref[i:i + 1, :] = idx
        acc = acc + jnp.sum(m)

    first = jnp.logical_and(pl.program_id(0) == 0, pl.program_id(1) == 0)
    acc2 = acc.reshape(1, 1)

    @pl.when(first)
    def _():
        loss_ref[:, :] = acc2

    @pl.when(jnp.logical_not(first))
    def _():
        loss_ref[:, :] = loss_ref[:, :] + acc2


def kernel(z, codebooks):
    b, c, h, w = z.shape
    n = h * w
    ncb, K, dpc = codebooks.shape
    z3 = z.reshape(b, c, n)
    cbm2 = codebooks * (-2.0)
    wsq = jnp.sum(codebooks * codebooks, axis=2)[:, :, None]
    NT = 2048 if n % 2048 == 0 else n
    tpb = n // NT
    grid = (b, tpb)
    zq3, idx2, loss_acc = pl.pallas_call(
        _vq_block,
        grid=grid,
        in_specs=[
            pl.BlockSpec((1, c, NT), lambda bi, ti: (bi, 0, ti)),
            pl.BlockSpec((ncb, K, dpc), lambda bi, ti: (0, 0, 0)),
            pl.BlockSpec((ncb, K, dpc), lambda bi, ti: (0, 0, 0)),
            pl.BlockSpec((ncb, K, 1), lambda bi, ti: (0, 0, 0)),
        ],
        out_specs=[
            pl.BlockSpec((1, c, NT), lambda bi, ti: (bi, 0, ti)),
            pl.BlockSpec((ncb, NT), lambda bi, ti: (0, bi * tpb + ti)),
            pl.BlockSpec((1, 1), lambda bi, ti: (0, 0)),
        ],
        out_shape=[
            jax.ShapeDtypeStruct((b, c, n), jnp.float32),
            jax.ShapeDtypeStruct((ncb, b * n), jnp.int32),
            jax.ShapeDtypeStruct((1, 1), jnp.float32),
        ],
    )(z3, codebooks, cbm2, wsq)
    z_q = zq3.reshape(b, c, h, w)
    total_loss = loss_acc[0, 0] * (1.25 / (ncb * b * n * dpc))
    indices = tuple(idx2[i].reshape(b, h, w) for i in range(ncb))
    return (z_q, total_loss, *indices)
